# Initial kernel scaffold; baseline (speedup 1.0000x reference)
import jax
import jax.numpy as jnp
from jax.experimental import pallas as pl


def kernel(x, edge_index, edge_attr, W0, as0, ad0, We0, ae0, b0, W1, as1, ad1, We1, ae1, b1, U0, ub0, U1, ub1):
    raise NotImplementedError("write your pallas kernel here")


# speedup vs baseline: 1.0961x; 1.0961x over previous
import functools

import jax
import jax.numpy as jnp
from jax.experimental import pallas as pl

N = 10000
E = 160000
D = 256
DE = 16
HID = 256


def _leaky(v):
    return jnp.where(v >= 0, v, 0.2 * v)


def _mm_body(x_ref, w_ref, o_ref):
    o_ref[...] = jnp.dot(x_ref[...], w_ref[...],
                         preferred_element_type=jnp.float32)


def _mm(x, w, bm):
    M, K = x.shape
    C = w.shape[1]
    return pl.pallas_call(
        _mm_body,
        grid=(M // bm,),
        in_specs=[pl.BlockSpec((bm, K), lambda i: (i, 0)),
                  pl.BlockSpec((K, C), lambda i: (0, 0))],
        out_specs=pl.BlockSpec((bm, C), lambda i: (i, 0)),
        out_shape=jax.ShapeDtypeStruct((M, C), jnp.float32),
    )(x, w)


def _ab_body(x2_ref, gs_ref, gd_ref, wa_ref, wb_ref, wc_ref, a_ref, b_ref):
    a_ref[...] = (jnp.dot(x2_ref[...], wa_ref[...], preferred_element_type=jnp.float32)
                  + jnp.dot(gs_ref[...], wc_ref[...], preferred_element_type=jnp.float32))
    b_ref[...] = (jnp.dot(x2_ref[...], wb_ref[...], preferred_element_type=jnp.float32)
                  + jnp.dot(gd_ref[...], wc_ref[...], preferred_element_type=jnp.float32))


def _ab(x2, gs, gd, wa, wb, wc, bm):
    M = x2.shape[0]
    in_spec = pl.BlockSpec((bm, HID), lambda i: (i, 0))
    w_spec = pl.BlockSpec((HID, HID), lambda i: (0, 0))
    return pl.pallas_call(
        _ab_body,
        grid=(M // bm,),
        in_specs=[in_spec, in_spec, in_spec, w_spec, w_spec, w_spec],
        out_specs=(pl.BlockSpec((bm, HID), lambda i: (i, 0)),) * 2,
        out_shape=(jax.ShapeDtypeStruct((M, HID), jnp.float32),) * 2,
    )(x2, gs, gd, wa, wb, wc)


def _gat_sparse(h, as_n, ad_n, he_e, b, src, dst):
    he_loop = he_e.mean()
    l_e = _leaky(as_n[src] + ad_n[dst] + he_e)
    l_loop = _leaky(as_n + ad_n + he_loop)
    w = jnp.exp(l_e - l_loop[dst])
    s = 1.0 + jax.ops.segment_sum(w, dst, num_segments=N)
    num = h + jax.ops.segment_sum(h[src] * w[:, None], dst, num_segments=N)
    return num / (s[:, None] + 1e-16) + b


def kernel(x, edge_index, edge_attr, W0, as0, ad0, We0, ae0, b0,
           W1, as1, ad1, We1, ae1, b1, U0, ub0, U1, ub1):
    src, dst = edge_index[0], edge_index[1]

    we0 = We0 @ ae0
    we1 = We1 @ ae1
    U0a, U0b, U0c = U0[:HID], U0[HID:2 * HID], U0[2 * HID:]
    U1a, U1b, U1c = U1[:HID], U1[HID:2 * HID], U1[2 * HID:]
    C = U0c @ U1c
    dvec = ub0 @ U1c + ub1
    c1 = ub0 @ we1
    v1 = U0c @ we1
    S0 = jnp.zeros((HID, 128), jnp.float32).at[:, 0].set(W0 @ as0).at[:, 1].set(W0 @ ad0)
    Wbig0 = jnp.concatenate([W0, S0], axis=1)
    Svec1 = (jnp.zeros((HID, 128), jnp.float32)
             .at[:, 0].set(W1 @ as1).at[:, 1].set(W1 @ ad1)
             .at[:, 2].set(U0a @ we1).at[:, 3].set(U0b @ we1))
    Wbig1 = jnp.concatenate([W1, U0a, U0b, Svec1], axis=1)
    Wea = (jnp.zeros((DE, 384), jnp.float32)
           .at[:, 0].set(we0).at[:, 1].set(v1).at[:, 128:].set(C))
    bea = jnp.zeros((384,), jnp.float32).at[1].set(c1).at[128:].set(dvec)

    out0 = _mm(x, Wbig0, 1000)
    h0, as0_n, ad0_n = out0[:, :HID], out0[:, HID], out0[:, HID + 1]

    ea_out = _mm(edge_attr, Wea, 4000) + bea
    t0_e, t1_e, eac = ea_out[:, 0], ea_out[:, 1], ea_out[:, 128:]

    x1 = _gat_sparse(h0, as0_n, ad0_n, t0_e, b0, src, dst)

    out1 = _mm(x1, Wbig1, 1000)
    h1, Gs, Gd = out1[:, :HID], out1[:, HID:2 * HID], out1[:, 2 * HID:3 * HID]
    sc = out1[:, 3 * HID:]
    as1_n, ad1_n, p_n, q_n = sc[:, 0], sc[:, 1], sc[:, 2], sc[:, 3]

    he1 = p_n[src] + q_n[dst] + t1_e
    x2 = _gat_sparse(h1, as1_n, ad1_n, he1, b1, src, dst)

    A, B = _ab(x2, Gs, Gd, U1a, U1b, U1c, 1000)

    e2 = A[src] + B[dst] + eac
    return x2, e2


# speedup vs baseline: 1.1613x; 1.0595x over previous
import functools

import jax
import jax.numpy as jnp
from jax import lax
from jax.experimental import pallas as pl
from jax.experimental.pallas import tpu as pltpu
from jax.experimental.pallas import tpu_sc as plsc

N = 10000
E = 160000
D = 256
DE = 16
HID = 256

_MESH = plsc.VectorSubcoreMesh(core_axis_name="c", subcore_axis_name="s")
NW = 32


def _leaky(v):
    return jnp.where(v >= 0, v, 0.2 * v)


def _mm_body(x_ref, w_ref, o_ref):
    o_ref[...] = jnp.dot(x_ref[...], w_ref[...],
                         preferred_element_type=jnp.float32)


def _mm(x, w, bm):
    M, K = x.shape
    C = w.shape[1]
    return pl.pallas_call(
        _mm_body,
        grid=(M // bm,),
        in_specs=[pl.BlockSpec((bm, K), lambda i: (i, 0)),
                  pl.BlockSpec((K, C), lambda i: (0, 0))],
        out_specs=pl.BlockSpec((bm, C), lambda i: (i, 0)),
        out_shape=jax.ShapeDtypeStruct((M, C), jnp.float32),
    )(x, w)


def _ab_body(x2_ref, gs_ref, gd_ref, wa_ref, wb_ref, wc_ref, a_ref, b_ref):
    a_ref[...] = (jnp.dot(x2_ref[...], wa_ref[...], preferred_element_type=jnp.float32)
                  + jnp.dot(gs_ref[...], wc_ref[...], preferred_element_type=jnp.float32))
    b_ref[...] = (jnp.dot(x2_ref[...], wb_ref[...], preferred_element_type=jnp.float32)
                  + jnp.dot(gd_ref[...], wc_ref[...], preferred_element_type=jnp.float32))


def _ab(x2, gs, gd, wa, wb, wc, bm):
    M = x2.shape[0]
    in_spec = pl.BlockSpec((bm, HID), lambda i: (i, 0))
    w_spec = pl.BlockSpec((HID, HID), lambda i: (0, 0))
    return pl.pallas_call(
        _ab_body,
        grid=(M // bm,),
        in_specs=[in_spec, in_spec, in_spec, w_spec, w_spec, w_spec],
        out_specs=(pl.BlockSpec((bm, HID), lambda i: (i, 0)),) * 2,
        out_shape=(jax.ShapeDtypeStruct((M, HID), jnp.float32),) * 2,
    )(x2, gs, gd, wa, wb, wc)


def _ea_body(ea_ref, w_ref, bias_ref, t_ref, eac_ref):
    acc = (jnp.dot(ea_ref[...], w_ref[...], preferred_element_type=jnp.float32)
           + bias_ref[...])
    t_ref[...] = acc[:, :128]
    eac_ref[...] = acc[:, 128:]


def _ea_mm(ea, w, bias, bm):
    return pl.pallas_call(
        _ea_body,
        grid=(E // bm,),
        in_specs=[pl.BlockSpec((bm, DE), lambda i: (i, 0)),
                  pl.BlockSpec((DE, 384), lambda i: (0, 0)),
                  pl.BlockSpec((1, 384), lambda i: (0, 0))],
        out_specs=(pl.BlockSpec((bm, 128), lambda i: (i, 0)),
                   pl.BlockSpec((bm, HID), lambda i: (i, 0))),
        out_shape=(jax.ShapeDtypeStruct((E, 128), jnp.float32),
                   jax.ShapeDtypeStruct((E, HID), jnp.float32)),
    )(ea, w, bias)


_EW = E // NW
_ECH = 200
_ENCH = _EW // _ECH


@functools.partial(
    pl.kernel,
    out_type=jax.ShapeDtypeStruct((E, HID), jnp.float32),
    mesh=_MESH,
    scratch_types=[
        pltpu.VMEM((_EW,), jnp.int32),
        pltpu.VMEM((_EW,), jnp.int32),
        pltpu.VMEM((_ECH, HID), jnp.float32),
        pltpu.VMEM((_ECH, HID), jnp.float32),
        pltpu.SemaphoreType.DMA,
        pltpu.SemaphoreType.DMA,
    ],
)
def _e2_assemble(a_hbm, b_hbm, eac_hbm, src_hbm, dst_hbm, out_hbm,
                 src_v, dst_v, bufa, bufb, sema, semb):
    w = lax.axis_index("s") * 2 + lax.axis_index("c")
    ebase = w * _EW
    pltpu.sync_copy(src_hbm.at[pl.ds(ebase, _EW)], src_v)
    pltpu.sync_copy(dst_hbm.at[pl.ds(ebase, _EW)], dst_v)

    def add_into_a(e, _):
        for v in range(HID // 16):
            sl = pl.ds(v * 16, 16)
            bufa[e, sl] = bufa[e, sl] + bufb[e, sl]
        return 0

    def chunk(k, _):
        cb = k * _ECH
        cpa = pltpu.async_copy(a_hbm.at[src_v.at[pl.ds(cb, _ECH)]], bufa, sema)
        cpb = pltpu.async_copy(b_hbm.at[dst_v.at[pl.ds(cb, _ECH)]], bufb, semb)
        cpa.wait()
        cpb.wait()
        lax.fori_loop(0, _ECH, add_into_a, 0)
        pltpu.sync_copy(eac_hbm.at[pl.ds(ebase + cb, _ECH)], bufb)
        lax.fori_loop(0, _ECH, add_into_a, 0)
        pltpu.sync_copy(bufa, out_hbm.at[pl.ds(ebase + cb, _ECH)])
        return 0

    lax.fori_loop(0, _ENCH, chunk, 0)


def _gat_sparse(h, as_n, ad_n, he_e, b, src, dst):
    he_loop = he_e.mean()
    l_e = _leaky(as_n[src] + ad_n[dst] + he_e)
    l_loop = _leaky(as_n + ad_n + he_loop)
    w = jnp.exp(l_e - l_loop[dst])
    s = 1.0 + jax.ops.segment_sum(w, dst, num_segments=N)
    num = h + jax.ops.segment_sum(h[src] * w[:, None], dst, num_segments=N)
    return num / (s[:, None] + 1e-16) + b


def kernel(x, edge_index, edge_attr, W0, as0, ad0, We0, ae0, b0,
           W1, as1, ad1, We1, ae1, b1, U0, ub0, U1, ub1):
    src, dst = edge_index[0], edge_index[1]

    we0 = We0 @ ae0
    we1 = We1 @ ae1
    U0a, U0b, U0c = U0[:HID], U0[HID:2 * HID], U0[2 * HID:]
    U1a, U1b, U1c = U1[:HID], U1[HID:2 * HID], U1[2 * HID:]
    C = U0c @ U1c
    dvec = ub0 @ U1c + ub1
    c1 = ub0 @ we1
    v1 = U0c @ we1
    S0 = jnp.zeros((HID, 128), jnp.float32).at[:, 0].set(W0 @ as0).at[:, 1].set(W0 @ ad0)
    Wbig0 = jnp.concatenate([W0, S0], axis=1)
    Svec1 = (jnp.zeros((HID, 128), jnp.float32)
             .at[:, 0].set(W1 @ as1).at[:, 1].set(W1 @ ad1)
             .at[:, 2].set(U0a @ we1).at[:, 3].set(U0b @ we1))
    Wbig1 = jnp.concatenate([W1, U0a, U0b, Svec1], axis=1)
    Wea = (jnp.zeros((DE, 384), jnp.float32)
           .at[:, 0].set(we0).at[:, 1].set(v1).at[:, 128:].set(C))
    bea = jnp.zeros((384,), jnp.float32).at[1].set(c1).at[128:].set(dvec)

    out0 = _mm(x, Wbig0, 1000)
    h0, as0_n, ad0_n = out0[:, :HID], out0[:, HID], out0[:, HID + 1]

    tcols, eac = _ea_mm(edge_attr, Wea, bea[None, :], 4000)
    t0_e, t1_e = tcols[:, 0], tcols[:, 1]

    x1 = _gat_sparse(h0, as0_n, ad0_n, t0_e, b0, src, dst)

    out1 = _mm(x1, Wbig1, 1000)
    h1, Gs, Gd = out1[:, :HID], out1[:, HID:2 * HID], out1[:, 2 * HID:3 * HID]
    sc = out1[:, 3 * HID:]
    as1_n, ad1_n, p_n, q_n = sc[:, 0], sc[:, 1], sc[:, 2], sc[:, 3]

    he1 = p_n[src] + q_n[dst] + t1_e
    x2 = _gat_sparse(h1, as1_n, ad1_n, he1, b1, src, dst)

    A, B = _ab(x2, Gs, Gd, U1a, U1b, U1c, 1000)

    e2 = _e2_assemble(A, B, eac, src, dst)
    return x2, e2


# speedup vs baseline: 5.5968x; 4.8194x over previous
import functools

import jax
import jax.numpy as jnp
from jax import lax
from jax.experimental import pallas as pl
from jax.experimental.pallas import tpu as pltpu
from jax.experimental.pallas import tpu_sc as plsc

N = 10000
E = 160000
D = 256
DE = 16
HID = 256
NP = 10240
NPH = 10752

_MESH = plsc.VectorSubcoreMesh(core_axis_name="c", subcore_axis_name="s")
NW = 32


def _leaky(v):
    return jnp.where(v >= 0, v, 0.2 * v)


def _st1_body(x_ref, w_ref, h_ref, sc_ref):
    acc = jnp.dot(x_ref[...], w_ref[...], preferred_element_type=jnp.float32)
    h_ref[0] = acc[:, :128]
    h_ref[1] = acc[:, 128:256]
    sc_ref[...] = acc[:, 256:]


def _stage1(x, w, bm):
    return pl.pallas_call(
        _st1_body,
        grid=(N // bm,),
        in_specs=[pl.BlockSpec((bm, HID), lambda i: (i, 0)),
                  pl.BlockSpec((HID, 384), lambda i: (0, 0))],
        out_specs=(pl.BlockSpec((2, bm, 128), lambda i: (0, i, 0)),
                   pl.BlockSpec((bm, 128), lambda i: (i, 0))),
        out_shape=(jax.ShapeDtypeStruct((2, NPH, 128), jnp.float32),
                   jax.ShapeDtypeStruct((N, 128), jnp.float32)),
    )(x, w)


def _st3_body(x_ref, w_ref, h_ref, gs_ref, gd_ref, sc_ref):
    xb = jnp.concatenate([x_ref[0], x_ref[1]], axis=1)
    acc = jnp.dot(xb, w_ref[...], preferred_element_type=jnp.float32)
    h_ref[0] = acc[:, :128]
    h_ref[1] = acc[:, 128:256]
    gs_ref[...] = acc[:, 256:512]
    gd_ref[...] = acc[:, 512:768]
    sc_ref[...] = acc[:, 768:]


def _stage3(x1, w, bm):
    return pl.pallas_call(
        _st3_body,
        grid=(N // bm,),
        in_specs=[pl.BlockSpec((2, bm, 128), lambda i: (0, i, 0)),
                  pl.BlockSpec((HID, 896), lambda i: (0, 0))],
        out_specs=(pl.BlockSpec((2, bm, 128), lambda i: (0, i, 0)),
                   pl.BlockSpec((bm, HID), lambda i: (i, 0)),
                   pl.BlockSpec((bm, HID), lambda i: (i, 0)),
                   pl.BlockSpec((bm, 128), lambda i: (i, 0))),
        out_shape=(jax.ShapeDtypeStruct((2, NPH, 128), jnp.float32),
                   jax.ShapeDtypeStruct((N, HID), jnp.float32),
                   jax.ShapeDtypeStruct((N, HID), jnp.float32),
                   jax.ShapeDtypeStruct((N, 128), jnp.float32)),
    )(x1, w)


def _st4_body(x2_ref, gs_ref, gd_ref, wa_ref, wb_ref, wc_ref,
              a_ref, b_ref, x2m_ref):
    xb = jnp.concatenate([x2_ref[0], x2_ref[1]], axis=1)
    a_ref[...] = (jnp.dot(xb, wa_ref[...], preferred_element_type=jnp.float32)
                  + jnp.dot(gs_ref[...], wc_ref[...], preferred_element_type=jnp.float32))
    b_ref[...] = (jnp.dot(xb, wb_ref[...], preferred_element_type=jnp.float32)
                  + jnp.dot(gd_ref[...], wc_ref[...], preferred_element_type=jnp.float32))
    x2m_ref[...] = xb


def _stage4(x2, gs, gd, wa, wb, wc, bm):
    w_spec = pl.BlockSpec((HID, HID), lambda i: (0, 0))
    m_spec = pl.BlockSpec((bm, HID), lambda i: (i, 0))
    return pl.pallas_call(
        _st4_body,
        grid=(N // bm,),
        in_specs=[pl.BlockSpec((2, bm, 128), lambda i: (0, i, 0)),
                  m_spec, m_spec, w_spec, w_spec, w_spec],
        out_specs=(m_spec, m_spec, m_spec),
        out_shape=(jax.ShapeDtypeStruct((N, HID), jnp.float32),) * 3,
    )(x2, gs, gd, wa, wb, wc)


def _ea_body(ea_ref, w_ref, bias_ref, t_ref, eac_ref):
    acc = (jnp.dot(ea_ref[...], w_ref[...], preferred_element_type=jnp.float32)
           + bias_ref[...])
    t_ref[...] = acc[:, :128]
    eac_ref[...] = acc[:, 128:]


def _ea_mm(ea, w, bias, bm):
    return pl.pallas_call(
        _ea_body,
        grid=(E // bm,),
        in_specs=[pl.BlockSpec((bm, DE), lambda i: (i, 0)),
                  pl.BlockSpec((DE, 384), lambda i: (0, 0)),
                  pl.BlockSpec((1, 384), lambda i: (0, 0))],
        out_specs=(pl.BlockSpec((bm, 128), lambda i: (i, 0)),
                   pl.BlockSpec((bm, HID), lambda i: (i, 0))),
        out_shape=(jax.ShapeDtypeStruct((E, 128), jnp.float32),
                   jax.ShapeDtypeStruct((E, HID), jnp.float32)),
    )(ea, w, bias)


_EC = E // 16
_RCH = 80
_RNCH = _EC // _RCH
_NH = 3584
_NPARTS = 3
_TRASH = _NH
_NHS = _NH // 16
_NPOUT = 17408
_OSL = 112


def _make_gat_sc(has_pq):
    scratch = [
        pltpu.VMEM((NP,), jnp.float32),
        pltpu.VMEM((NP,), jnp.float32),
    ]
    if has_pq:
        scratch += [pltpu.VMEM((NP,), jnp.float32),
                    pltpu.VMEM((NP,), jnp.float32)]
    scratch += [
        pltpu.VMEM((_EC,), jnp.int32),
        pltpu.VMEM((_EC,), jnp.int32),
        pltpu.VMEM((_RCH,), jnp.int32),
        pltpu.VMEM((_EC,), jnp.float32),
        pltpu.VMEM((_RCH, 128), jnp.float32),
        pltpu.VMEM((_OSL, 128), jnp.float32),
        pltpu.VMEM((_NHS,), jnp.float32),
        pltpu.VMEM((16, 16), jnp.float32),
        pltpu.VMEM((16,), jnp.float32),
        pltpu.VMEM((128,), jnp.float32),
        pltpu.VMEM_SHARED((_NH + 8, 128), jnp.float32),
        pltpu.VMEM_SHARED((_NH + 8,), jnp.float32),
        pltpu.VMEM_SHARED((16, 16), jnp.float32),
        pltpu.SemaphoreType.DMA,
    ]

    def body(*args):
        if has_pq:
            (h2_hbm, na_hbm, nd_hbm, ga_hbm, gd_hbm, t_hbm, src_hbm, dst_hbm,
             b2_hbm, out_hbm,
             na_v, nd_v, ga_v, gd_v, src_v, dst_v, idxbuf, tlw_v, rowbuf,
             outbuf, sbuf, red_v, misc_v, b_v, num_sh, s_sh, red_sh, sem) = args
        else:
            (h2_hbm, na_hbm, nd_hbm, t_hbm, src_hbm, dst_hbm,
             b2_hbm, out_hbm,
             na_v, nd_v, src_v, dst_v, idxbuf, tlw_v, rowbuf,
             outbuf, sbuf, red_v, misc_v, b_v, num_sh, s_sh, red_sh, sem) = args

        c = lax.axis_index("c")
        sid = lax.axis_index("s")
        ebase = sid * _EC

        pltpu.sync_copy(na_hbm, na_v)
        pltpu.sync_copy(nd_hbm, nd_v)
        if has_pq:
            pltpu.sync_copy(ga_hbm, ga_v)
            pltpu.sync_copy(gd_hbm, gd_v)
        pltpu.sync_copy(src_hbm.at[pl.ds(ebase, _EC)], src_v)
        pltpu.sync_copy(dst_hbm.at[pl.ds(ebase, _EC)], dst_v)
        pltpu.sync_copy(t_hbm.at[pl.ds(ebase, _EC)], tlw_v)
        pltpu.sync_copy(b2_hbm.at[c], b_v)

        def p_logit(i, acc):
            sl = pl.ds(i * 16, 16)
            s16 = src_v[sl]
            d16 = dst_v[sl]
            he = tlw_v[sl]
            if has_pq:
                he = (he + plsc.load_gather(ga_v, [s16])
                      + plsc.load_gather(gd_v, [d16]))
            lo = plsc.load_gather(na_v, [s16]) + plsc.load_gather(nd_v, [d16]) + he
            tlw_v[sl] = _leaky(lo)
            return acc + he

        acc = lax.fori_loop(0, _EC // 16, p_logit,
                            jnp.zeros((16,), jnp.float32))
        misc_v[...] = acc
        pltpu.sync_copy(misc_v, red_sh.at[sid])
        plsc.subcore_barrier()
        pltpu.sync_copy(red_sh, red_v)
        tot = jnp.zeros((16,), jnp.float32)
        for j in range(16):
            tot = tot + red_v[j]
        he_loop = jnp.sum(tot * (1.0 / E))

        def p_w(i, _):
            sl = pl.ds(i * 16, 16)
            d16 = dst_v[sl]
            llv = _leaky(plsc.load_gather(na_v, [d16])
                         + plsc.load_gather(nd_v, [d16]) + he_loop)
            tlw_v[sl] = jnp.exp(tlw_v[sl] - llv)
            return 0

        lax.fori_loop(0, _EC // 16, p_w, 0)

        for half in range(_NPARTS):
            lo = half * _NH
            nbase = sid * _NHS

            pltpu.sync_copy(h2_hbm.at[c, pl.ds(lo + nbase, _NHS)],
                            num_sh.at[pl.ds(nbase, _NHS)])

            def p_one(i, _):
                sbuf[pl.ds(i * 16, 16)] = jnp.full((16,), 1.0, jnp.float32)
                return 0

            lax.fori_loop(0, _NHS // 16, p_one, 0)
            pltpu.sync_copy(sbuf, s_sh.at[pl.ds(nbase, _NHS)])
            plsc.subcore_barrier()

            def p_rows(k, _):
                cb = k * _RCH
                cp = pltpu.async_copy(
                    h2_hbm.at[c].at[src_v.at[pl.ds(cb, _RCH)]], rowbuf, sem)

                def mkidx(g, _):
                    d16 = dst_v[pl.ds(cb + g * 16, 16)]
                    inh = (d16 >= lo) & (d16 < lo + _NH)
                    idxbuf[pl.ds(g * 16, 16)] = jnp.where(inh, d16 - lo, _TRASH)
                    return 0

                lax.fori_loop(0, _RCH // 16, mkidx, 0)
                pltpu.sync_copy(tlw_v.at[pl.ds(cb, _RCH)],
                                s_sh.at[idxbuf], add=True)
                cp.wait()

                def rowscale(g, _):
                    w16 = tlw_v[pl.ds(cb + g * 16, 16)]
                    for j in range(16):
                        e = g * 16 + j
                        wsc = w16[j]
                        for v in range(8):
                            sl2 = pl.ds(v * 16, 16)
                            rowbuf[e, sl2] = rowbuf[e, sl2] * wsc
                    return 0

                lax.fori_loop(0, _RCH // 16, rowscale, 0)
                pltpu.sync_copy(rowbuf, num_sh.at[idxbuf], add=True)
                return 0

            lax.fori_loop(0, _RNCH, p_rows, 0)
            plsc.subcore_barrier()

            pltpu.sync_copy(s_sh.at[pl.ds(nbase, _NHS)], sbuf)

            def p_inv(i, _):
                sl = pl.ds(i * 16, 16)
                sbuf[sl] = 1.0 / (sbuf[sl] + 1e-16)
                return 0

            lax.fori_loop(0, _NHS // 16, p_inv, 0)

            def p_out(jj, _):
                nb2 = jj * _OSL
                pltpu.sync_copy(num_sh.at[pl.ds(nbase + nb2, _OSL)], outbuf)

                def orow(g, _):
                    inv16 = sbuf[pl.ds(nb2 + g * 16, 16)]
                    for j in range(16):
                        i = g * 16 + j
                        inv = inv16[j]
                        for v in range(8):
                            sl2 = pl.ds(v * 16, 16)
                            outbuf[i, sl2] = outbuf[i, sl2] * inv + b_v[sl2]
                    return 0

                lax.fori_loop(0, _OSL // 16, orow, 0)
                pltpu.sync_copy(outbuf,
                                out_hbm.at[c, pl.ds(lo + nbase + nb2, _OSL)])
                return 0

            lax.fori_loop(0, _NHS // _OSL, p_out, 0)
            plsc.subcore_barrier()

    return functools.partial(
        pl.kernel,
        out_type=jax.ShapeDtypeStruct((2, _NPOUT, 128), jnp.float32),
        mesh=_MESH,
        scratch_types=scratch,
        compiler_params=pltpu.CompilerParams(needs_layout_passes=False),
    )(body)


_gat_sc0 = _make_gat_sc(False)
_gat_sc1 = _make_gat_sc(True)


_EW = E // NW
_ECH = 200
_ENCH = _EW // _ECH


@functools.partial(
    pl.kernel,
    out_type=jax.ShapeDtypeStruct((E, HID), jnp.float32),
    mesh=_MESH,
    scratch_types=[
        pltpu.VMEM((_EW,), jnp.int32),
        pltpu.VMEM((_EW,), jnp.int32),
        pltpu.VMEM((_ECH, HID), jnp.float32),
        pltpu.VMEM((_ECH, HID), jnp.float32),
        pltpu.SemaphoreType.DMA,
        pltpu.SemaphoreType.DMA,
    ],
)
def _e2_assemble(a_hbm, b_hbm, eac_hbm, src_hbm, dst_hbm, out_hbm,
                 src_v, dst_v, bufa, bufb, sema, semb):
    w = lax.axis_index("s") * 2 + lax.axis_index("c")
    ebase = w * _EW
    pltpu.sync_copy(src_hbm.at[pl.ds(ebase, _EW)], src_v)
    pltpu.sync_copy(dst_hbm.at[pl.ds(ebase, _EW)], dst_v)

    def add_into_a(e, _):
        for v in range(HID // 16):
            sl = pl.ds(v * 16, 16)
            bufa[e, sl] = bufa[e, sl] + bufb[e, sl]
        return 0

    def chunk(k, _):
        cb = k * _ECH
        cpa = pltpu.async_copy(a_hbm.at[src_v.at[pl.ds(cb, _ECH)]], bufa, sema)
        cpb = pltpu.async_copy(b_hbm.at[dst_v.at[pl.ds(cb, _ECH)]], bufb, semb)
        cpa.wait()
        cpb.wait()
        lax.fori_loop(0, _ECH, add_into_a, 0)
        pltpu.sync_copy(eac_hbm.at[pl.ds(ebase + cb, _ECH)], bufb)
        lax.fori_loop(0, _ECH, add_into_a, 0)
        pltpu.sync_copy(bufa, out_hbm.at[pl.ds(ebase + cb, _ECH)])
        return 0

    lax.fori_loop(0, _ENCH, chunk, 0)


def _pad_n(v):
    return jnp.pad(v, (0, NP - N))


def kernel(x, edge_index, edge_attr, W0, as0, ad0, We0, ae0, b0,
           W1, as1, ad1, We1, ae1, b1, U0, ub0, U1, ub1):
    src, dst = edge_index[0], edge_index[1]

    we0 = We0 @ ae0
    we1 = We1 @ ae1
    U0a, U0b, U0c = U0[:HID], U0[HID:2 * HID], U0[2 * HID:]
    U1a, U1b, U1c = U1[:HID], U1[HID:2 * HID], U1[2 * HID:]
    C = U0c @ U1c
    dvec = ub0 @ U1c + ub1
    c1 = ub0 @ we1
    v1 = U0c @ we1
    S0 = jnp.zeros((HID, 128), jnp.float32).at[:, 0].set(W0 @ as0).at[:, 1].set(W0 @ ad0)
    Wbig0 = jnp.concatenate([W0, S0], axis=1)
    Svec1 = (jnp.zeros((HID, 128), jnp.float32)
             .at[:, 0].set(W1 @ as1).at[:, 1].set(W1 @ ad1)
             .at[:, 2].set(U0a @ we1).at[:, 3].set(U0b @ we1))
    Wbig1 = jnp.concatenate([W1, U0a, U0b, Svec1], axis=1)
    Wea = (jnp.zeros((DE, 384), jnp.float32)
           .at[:, 0].set(we0).at[:, 1].set(v1).at[:, 128:].set(C))
    bea = jnp.zeros((384,), jnp.float32).at[1].set(c1).at[128:].set(dvec)

    h0s, scal0 = _stage1(x, Wbig0, 1000)
    na0, nd0 = _pad_n(scal0[:, 0]), _pad_n(scal0[:, 1])

    tcols, eac = _ea_mm(edge_attr, Wea, bea[None, :], 4000)
    t0_e, t1_e = tcols[:, 0], tcols[:, 1]

    x1s = _gat_sc0(h0s, na0, nd0, t0_e, src, dst,
                   b0.reshape(2, 128))

    h1s, Gs, Gd, scal1 = _stage3(x1s, Wbig1, 1000)
    na1, nd1 = _pad_n(scal1[:, 0]), _pad_n(scal1[:, 1])
    ga1, gd1 = _pad_n(scal1[:, 2]), _pad_n(scal1[:, 3])

    x2s = _gat_sc1(h1s, na1, nd1, ga1, gd1, t1_e, src, dst,
                   b1.reshape(2, 128))

    A, B, x2 = _stage4(x2s, Gs, Gd, U1a, U1b, U1c, 1000)

    e2 = _e2_assemble(A, B, eac, src, dst)
    return x2, e2


# speedup vs baseline: 5.6003x; 1.0006x over previous
import functools

import jax
import jax.numpy as jnp
from jax import lax
from jax.experimental import pallas as pl
from jax.experimental.pallas import tpu as pltpu
from jax.experimental.pallas import tpu_sc as plsc

N = 10000
E = 160000
D = 256
DE = 16
HID = 256
NP = 10240
NPH = 10752

_MESH = plsc.VectorSubcoreMesh(core_axis_name="c", subcore_axis_name="s")
NW = 32


def _leaky(v):
    return jnp.where(v >= 0, v, 0.2 * v)


def _st1_body(x_ref, w_ref, h_ref, sc_ref):
    acc = jnp.dot(x_ref[...], w_ref[...], preferred_element_type=jnp.float32)
    h_ref[0] = acc[:, :128]
    h_ref[1] = acc[:, 128:256]
    sc_ref[...] = acc[:, 256:]


def _stage1(x, w, bm):
    return pl.pallas_call(
        _st1_body,
        grid=(N // bm,),
        in_specs=[pl.BlockSpec((bm, HID), lambda i: (i, 0)),
                  pl.BlockSpec((HID, 384), lambda i: (0, 0))],
        out_specs=(pl.BlockSpec((2, bm, 128), lambda i: (0, i, 0)),
                   pl.BlockSpec((bm, 128), lambda i: (i, 0))),
        out_shape=(jax.ShapeDtypeStruct((2, NPH, 128), jnp.float32),
                   jax.ShapeDtypeStruct((N, 128), jnp.float32)),
    )(x, w)


def _st3_body(x_ref, w_ref, h_ref, gs_ref, gd_ref, sc_ref):
    xb = jnp.concatenate([x_ref[0], x_ref[1]], axis=1)
    acc = jnp.dot(xb, w_ref[...], preferred_element_type=jnp.float32)
    h_ref[0] = acc[:, :128]
    h_ref[1] = acc[:, 128:256]
    gs_ref[...] = acc[:, 256:512]
    gd_ref[...] = acc[:, 512:768]
    sc_ref[...] = acc[:, 768:]


def _stage3(x1, w, bm):
    return pl.pallas_call(
        _st3_body,
        grid=(N // bm,),
        in_specs=[pl.BlockSpec((2, bm, 128), lambda i: (0, i, 0)),
                  pl.BlockSpec((HID, 896), lambda i: (0, 0))],
        out_specs=(pl.BlockSpec((2, bm, 128), lambda i: (0, i, 0)),
                   pl.BlockSpec((bm, HID), lambda i: (i, 0)),
                   pl.BlockSpec((bm, HID), lambda i: (i, 0)),
                   pl.BlockSpec((bm, 128), lambda i: (i, 0))),
        out_shape=(jax.ShapeDtypeStruct((2, NPH, 128), jnp.float32),
                   jax.ShapeDtypeStruct((N, HID), jnp.float32),
                   jax.ShapeDtypeStruct((N, HID), jnp.float32),
                   jax.ShapeDtypeStruct((N, 128), jnp.float32)),
    )(x1, w)


def _st4_body(x2_ref, gs_ref, gd_ref, wa_ref, wb_ref, wc_ref,
              a_ref, b_ref, x2m_ref):
    xb = jnp.concatenate([x2_ref[0], x2_ref[1]], axis=1)
    a_ref[...] = (jnp.dot(xb, wa_ref[...], preferred_element_type=jnp.float32)
                  + jnp.dot(gs_ref[...], wc_ref[...], preferred_element_type=jnp.float32))
    b_ref[...] = (jnp.dot(xb, wb_ref[...], preferred_element_type=jnp.float32)
                  + jnp.dot(gd_ref[...], wc_ref[...], preferred_element_type=jnp.float32))
    x2m_ref[...] = xb


def _stage4(x2, gs, gd, wa, wb, wc, bm):
    w_spec = pl.BlockSpec((HID, HID), lambda i: (0, 0))
    m_spec = pl.BlockSpec((bm, HID), lambda i: (i, 0))
    return pl.pallas_call(
        _st4_body,
        grid=(N // bm,),
        in_specs=[pl.BlockSpec((2, bm, 128), lambda i: (0, i, 0)),
                  m_spec, m_spec, w_spec, w_spec, w_spec],
        out_specs=(m_spec, m_spec, m_spec),
        out_shape=(jax.ShapeDtypeStruct((N, HID), jnp.float32),) * 3,
    )(x2, gs, gd, wa, wb, wc)


def _ea_body(ea_ref, w_ref, bias_ref, t_ref, eac_ref):
    acc = (jnp.dot(ea_ref[...], w_ref[...], preferred_element_type=jnp.float32)
           + bias_ref[...])
    t_ref[...] = acc[:, :128]
    eac_ref[...] = acc[:, 128:]


def _ea_mm(ea, w, bias, bm):
    return pl.pallas_call(
        _ea_body,
        grid=(E // bm,),
        in_specs=[pl.BlockSpec((bm, DE), lambda i: (i, 0)),
                  pl.BlockSpec((DE, 384), lambda i: (0, 0)),
                  pl.BlockSpec((1, 384), lambda i: (0, 0))],
        out_specs=(pl.BlockSpec((bm, 128), lambda i: (i, 0)),
                   pl.BlockSpec((bm, HID), lambda i: (i, 0))),
        out_shape=(jax.ShapeDtypeStruct((E, 128), jnp.float32),
                   jax.ShapeDtypeStruct((E, HID), jnp.float32)),
    )(ea, w, bias)


_EC = E // 16
_RCH = 80
_RNCH = _EC // _RCH
_NPOUT = 17408


def _make_gat_sc(has_pq):
    nh = 3584
    nparts = 3
    nhs = nh // 16
    osl = nhs // 2
    scratch = [
        pltpu.VMEM((NP,), jnp.float32),
        pltpu.VMEM((NP,), jnp.float32),
    ]
    if has_pq:
        scratch += [pltpu.VMEM((NP,), jnp.float32),
                    pltpu.VMEM((NP,), jnp.float32)]
    scratch += [
        pltpu.VMEM((_EC,), jnp.int32),
        pltpu.VMEM((_EC,), jnp.int32),
        pltpu.VMEM((_RCH,), jnp.int32),
        pltpu.VMEM((_RCH,), jnp.int32),
        pltpu.VMEM((_EC,), jnp.float32),
        pltpu.VMEM((_RCH, 128), jnp.float32),
        pltpu.VMEM((_RCH, 128), jnp.float32),
        pltpu.VMEM((osl, 128), jnp.float32),
        pltpu.VMEM((nhs,), jnp.float32),
        pltpu.VMEM((16, 16), jnp.float32),
        pltpu.VMEM((16,), jnp.float32),
        pltpu.VMEM((128,), jnp.float32),
        pltpu.VMEM_SHARED((nh + 8, 128), jnp.float32),
        pltpu.VMEM_SHARED((nh + 8,), jnp.float32),
        pltpu.VMEM_SHARED((16, 16), jnp.float32),
        pltpu.SemaphoreType.DMA,
        pltpu.SemaphoreType.DMA,
    ]

    def body(*args):
        if has_pq:
            (h2_hbm, na_hbm, nd_hbm, ga_hbm, gd_hbm, t_hbm, src_hbm, dst_hbm,
             b2_hbm, out_hbm,
             na_v, nd_v, ga_v, gd_v, src_v, dst_v, idxA, idxB, tlw_v,
             rowA, rowB, outbuf, sbuf, red_v, misc_v, b_v,
             num_sh, s_sh, red_sh, semA, semB) = args
        else:
            (h2_hbm, na_hbm, nd_hbm, t_hbm, src_hbm, dst_hbm,
             b2_hbm, out_hbm,
             na_v, nd_v, src_v, dst_v, idxA, idxB, tlw_v,
             rowA, rowB, outbuf, sbuf, red_v, misc_v, b_v,
             num_sh, s_sh, red_sh, semA, semB) = args

        c = lax.axis_index("c")
        sid = lax.axis_index("s")
        ebase = sid * _EC

        pltpu.sync_copy(na_hbm, na_v)
        pltpu.sync_copy(nd_hbm, nd_v)
        if has_pq:
            pltpu.sync_copy(ga_hbm, ga_v)
            pltpu.sync_copy(gd_hbm, gd_v)
        pltpu.sync_copy(src_hbm.at[pl.ds(ebase, _EC)], src_v)
        pltpu.sync_copy(dst_hbm.at[pl.ds(ebase, _EC)], dst_v)
        pltpu.sync_copy(t_hbm.at[pl.ds(ebase, _EC)], tlw_v)
        pltpu.sync_copy(b2_hbm.at[c], b_v)

        def p_logit(i, acc):
            sl = pl.ds(i * 16, 16)
            s16 = src_v[sl]
            d16 = dst_v[sl]
            he = tlw_v[sl]
            if has_pq:
                he = (he + plsc.load_gather(ga_v, [s16])
                      + plsc.load_gather(gd_v, [d16]))
            lo = plsc.load_gather(na_v, [s16]) + plsc.load_gather(nd_v, [d16]) + he
            tlw_v[sl] = _leaky(lo)
            return acc + he

        acc = lax.fori_loop(0, _EC // 16, p_logit,
                            jnp.zeros((16,), jnp.float32))
        misc_v[...] = acc
        pltpu.sync_copy(misc_v, red_sh.at[sid])
        plsc.subcore_barrier()
        pltpu.sync_copy(red_sh, red_v)
        tot = jnp.zeros((16,), jnp.float32)
        for j in range(16):
            tot = tot + red_v[j]
        he_loop = jnp.sum(tot * (1.0 / E))

        def p_w(i, _):
            sl = pl.ds(i * 16, 16)
            d16 = dst_v[sl]
            llv = _leaky(plsc.load_gather(na_v, [d16])
                         + plsc.load_gather(nd_v, [d16]) + he_loop)
            tlw_v[sl] = jnp.exp(tlw_v[sl] - llv)
            return 0

        lax.fori_loop(0, _EC // 16, p_w, 0)

        def mk_chunk(lo, idxbuf, rowbuf):
            def go(cb, cp):
                def mkidx(g, _):
                    d16 = dst_v[pl.ds(cb + g * 16, 16)]
                    inh = (d16 >= lo) & (d16 < lo + nh)
                    idxbuf[pl.ds(g * 16, 16)] = jnp.where(inh, d16 - lo, nh)
                    return 0

                lax.fori_loop(0, _RCH // 16, mkidx, 0)
                pltpu.sync_copy(tlw_v.at[pl.ds(cb, _RCH)],
                                s_sh.at[idxbuf], add=True)
                cp.wait()

                def rowscale(g, _):
                    w16 = tlw_v[pl.ds(cb + g * 16, 16)]
                    for j in range(16):
                        e = g * 16 + j
                        wsc = w16[j]
                        for v in range(8):
                            sl2 = pl.ds(v * 16, 16)
                            rowbuf[e, sl2] = rowbuf[e, sl2] * wsc
                    return 0

                lax.fori_loop(0, _RCH // 16, rowscale, 0)
                pltpu.sync_copy(rowbuf, num_sh.at[idxbuf], add=True)
            return go

        def gather(cb, rowbuf, sem):
            return pltpu.async_copy(
                h2_hbm.at[c].at[src_v.at[pl.ds(cb, _RCH)]], rowbuf, sem)

        for part in range(nparts):
            lo = part * nh
            nbase = sid * nhs

            pltpu.sync_copy(h2_hbm.at[c, pl.ds(lo + nbase, nhs)],
                            num_sh.at[pl.ds(nbase, nhs)])

            def p_one(i, _):
                sbuf[pl.ds(i * 16, 16)] = jnp.full((16,), 1.0, jnp.float32)
                return 0

            lax.fori_loop(0, nhs // 16, p_one, 0)
            pltpu.sync_copy(sbuf, s_sh.at[pl.ds(nbase, nhs)])
            plsc.subcore_barrier()

            goA = mk_chunk(lo, idxA, rowA)

            def p_rows(k, _):
                cb = k * _RCH
                goA(cb, gather(cb, rowA, semA))
                return 0

            lax.fori_loop(0, _RNCH, p_rows, 0)
            plsc.subcore_barrier()

            pltpu.sync_copy(s_sh.at[pl.ds(nbase, nhs)], sbuf)

            def p_inv(i, _):
                sl = pl.ds(i * 16, 16)
                sbuf[sl] = 1.0 / (sbuf[sl] + 1e-16)
                return 0

            lax.fori_loop(0, nhs // 16, p_inv, 0)

            def p_out(jj, _):
                nb2 = jj * osl
                pltpu.sync_copy(num_sh.at[pl.ds(nbase + nb2, osl)], outbuf)

                def orow(g, _):
                    inv16 = sbuf[pl.ds(nb2 + g * 16, 16)]
                    for j in range(16):
                        i = g * 16 + j
                        inv = inv16[j]
                        for v in range(8):
                            sl2 = pl.ds(v * 16, 16)
                            outbuf[i, sl2] = outbuf[i, sl2] * inv + b_v[sl2]
                    return 0

                lax.fori_loop(0, osl // 16, orow, 0)
                pltpu.sync_copy(outbuf,
                                out_hbm.at[c, pl.ds(lo + nbase + nb2, osl)])
                return 0

            lax.fori_loop(0, nhs // osl, p_out, 0)
            plsc.subcore_barrier()

    return functools.partial(
        pl.kernel,
        out_type=jax.ShapeDtypeStruct((2, _NPOUT, 128), jnp.float32),
        mesh=_MESH,
        scratch_types=scratch,
        compiler_params=pltpu.CompilerParams(needs_layout_passes=False),
    )(body)


_gat_sc0 = _make_gat_sc(False)
_gat_sc1 = _make_gat_sc(True)


_EW = E // NW
_ECH = 200
_ENCH = _EW // _ECH


@functools.partial(
    pl.kernel,
    out_type=jax.ShapeDtypeStruct((E, HID), jnp.float32),
    mesh=_MESH,
    scratch_types=[
        pltpu.VMEM((_EW,), jnp.int32),
        pltpu.VMEM((_EW,), jnp.int32),
        pltpu.VMEM((_ECH, HID), jnp.float32),
        pltpu.VMEM((_ECH, HID), jnp.float32),
        pltpu.SemaphoreType.DMA,
        pltpu.SemaphoreType.DMA,
    ],
)
def _e2_assemble(a_hbm, b_hbm, eac_hbm, src_hbm, dst_hbm, out_hbm,
                 src_v, dst_v, bufa, bufb, sema, semb):
    w = lax.axis_index("s") * 2 + lax.axis_index("c")
    ebase = w * _EW
    pltpu.sync_copy(src_hbm.at[pl.ds(ebase, _EW)], src_v)
    pltpu.sync_copy(dst_hbm.at[pl.ds(ebase, _EW)], dst_v)

    def add_into_a(e, _):
        for v in range(HID // 16):
            sl = pl.ds(v * 16, 16)
            bufa[e, sl] = bufa[e, sl] + bufb[e, sl]
        return 0

    def chunk(k, _):
        cb = k * _ECH
        cpa = pltpu.async_copy(a_hbm.at[src_v.at[pl.ds(cb, _ECH)]], bufa, sema)
        cpb = pltpu.async_copy(b_hbm.at[dst_v.at[pl.ds(cb, _ECH)]], bufb, semb)
        cpa.wait()
        cpb.wait()
        lax.fori_loop(0, _ECH, add_into_a, 0)
        pltpu.sync_copy(eac_hbm.at[pl.ds(ebase + cb, _ECH)], bufb)
        lax.fori_loop(0, _ECH, add_into_a, 0)
        pltpu.sync_copy(bufa, out_hbm.at[pl.ds(ebase + cb, _ECH)])
        return 0

    lax.fori_loop(0, _ENCH, chunk, 0)


def _pad_n(v):
    return jnp.pad(v, (0, NP - N))


def kernel(x, edge_index, edge_attr, W0, as0, ad0, We0, ae0, b0,
           W1, as1, ad1, We1, ae1, b1, U0, ub0, U1, ub1):
    src, dst = edge_index[0], edge_index[1]

    we0 = We0 @ ae0
    we1 = We1 @ ae1
    U0a, U0b, U0c = U0[:HID], U0[HID:2 * HID], U0[2 * HID:]
    U1a, U1b, U1c = U1[:HID], U1[HID:2 * HID], U1[2 * HID:]
    C = U0c @ U1c
    dvec = ub0 @ U1c + ub1
    c1 = ub0 @ we1
    v1 = U0c @ we1
    S0 = jnp.zeros((HID, 128), jnp.float32).at[:, 0].set(W0 @ as0).at[:, 1].set(W0 @ ad0)
    Wbig0 = jnp.concatenate([W0, S0], axis=1)
    Svec1 = (jnp.zeros((HID, 128), jnp.float32)
             .at[:, 0].set(W1 @ as1).at[:, 1].set(W1 @ ad1)
             .at[:, 2].set(U0a @ we1).at[:, 3].set(U0b @ we1))
    Wbig1 = jnp.concatenate([W1, U0a, U0b, Svec1], axis=1)
    Wea = (jnp.zeros((DE, 384), jnp.float32)
           .at[:, 0].set(we0).at[:, 1].set(v1).at[:, 128:].set(C))
    bea = jnp.zeros((384,), jnp.float32).at[1].set(c1).at[128:].set(dvec)

    h0s, scal0 = _stage1(x, Wbig0, 1000)
    na0, nd0 = _pad_n(scal0[:, 0]), _pad_n(scal0[:, 1])

    tcols, eac = _ea_mm(edge_attr, Wea, bea[None, :], 4000)
    t0_e, t1_e = tcols[:, 0], tcols[:, 1]

    x1s = _gat_sc0(h0s, na0, nd0, t0_e, src, dst,
                   b0.reshape(2, 128))

    h1s, Gs, Gd, scal1 = _stage3(x1s, Wbig1, 1000)
    na1, nd1 = _pad_n(scal1[:, 0]), _pad_n(scal1[:, 1])
    ga1, gd1 = _pad_n(scal1[:, 2]), _pad_n(scal1[:, 3])

    x2s = _gat_sc1(h1s, na1, nd1, ga1, gd1, t1_e, src, dst,
                   b1.reshape(2, 128))

    A, B, x2 = _stage4(x2s, Gs, Gd, U1a, U1b, U1c, 1000)

    e2 = _e2_assemble(A, B, eac, src, dst)
    return x2, e2


# speedup vs baseline: 5.6041x; 1.0007x over previous
import functools

import jax
import jax.numpy as jnp
from jax import lax
from jax.experimental import pallas as pl
from jax.experimental.pallas import tpu as pltpu
from jax.experimental.pallas import tpu_sc as plsc

N = 10000
E = 160000
D = 256
DE = 16
HID = 256
NP = 10240
NPH = 10752

_MESH = plsc.VectorSubcoreMesh(core_axis_name="c", subcore_axis_name="s")
NW = 32


def _leaky(v):
    return jnp.where(v >= 0, v, 0.2 * v)


def _st1_body(x_ref, w_ref, h_ref, sc_ref):
    acc = jnp.dot(x_ref[...], w_ref[...], preferred_element_type=jnp.float32)
    h_ref[0] = acc[:, :128]
    h_ref[1] = acc[:, 128:256]
    sc_ref[...] = acc[:, 256:]


def _stage1(x, w, bm):
    return pl.pallas_call(
        _st1_body,
        grid=(N // bm,),
        in_specs=[pl.BlockSpec((bm, HID), lambda i: (i, 0)),
                  pl.BlockSpec((HID, 384), lambda i: (0, 0))],
        out_specs=(pl.BlockSpec((2, bm, 128), lambda i: (0, i, 0)),
                   pl.BlockSpec((bm, 128), lambda i: (i, 0))),
        out_shape=(jax.ShapeDtypeStruct((2, NPH, 128), jnp.float32),
                   jax.ShapeDtypeStruct((N, 128), jnp.float32)),
    )(x, w)


def _st3_body(x_ref, w_ref, h_ref, gs_ref, gd_ref, sc_ref):
    xb = jnp.concatenate([x_ref[0], x_ref[1]], axis=1)
    acc = jnp.dot(xb, w_ref[...], preferred_element_type=jnp.float32)
    h_ref[0] = acc[:, :128]
    h_ref[1] = acc[:, 128:256]
    gs_ref[...] = acc[:, 256:512]
    gd_ref[...] = acc[:, 512:768]
    sc_ref[...] = acc[:, 768:]


def _stage3(x1, w, bm):
    return pl.pallas_call(
        _st3_body,
        grid=(N // bm,),
        in_specs=[pl.BlockSpec((2, bm, 128), lambda i: (0, i, 0)),
                  pl.BlockSpec((HID, 896), lambda i: (0, 0))],
        out_specs=(pl.BlockSpec((2, bm, 128), lambda i: (0, i, 0)),
                   pl.BlockSpec((bm, HID), lambda i: (i, 0)),
                   pl.BlockSpec((bm, HID), lambda i: (i, 0)),
                   pl.BlockSpec((bm, 128), lambda i: (i, 0))),
        out_shape=(jax.ShapeDtypeStruct((2, NPH, 128), jnp.float32),
                   jax.ShapeDtypeStruct((N, HID), jnp.float32),
                   jax.ShapeDtypeStruct((N, HID), jnp.float32),
                   jax.ShapeDtypeStruct((N, 128), jnp.float32)),
    )(x1, w)


def _st4_body(x2_ref, gs_ref, gd_ref, wa_ref, wb_ref, wc_ref,
              a_ref, b_ref, x2m_ref):
    xb = jnp.concatenate([x2_ref[0], x2_ref[1]], axis=1)
    a_ref[...] = (jnp.dot(xb, wa_ref[...], preferred_element_type=jnp.float32)
                  + jnp.dot(gs_ref[...], wc_ref[...], preferred_element_type=jnp.float32))
    b_ref[...] = (jnp.dot(xb, wb_ref[...], preferred_element_type=jnp.float32)
                  + jnp.dot(gd_ref[...], wc_ref[...], preferred_element_type=jnp.float32))
    x2m_ref[...] = xb


def _stage4(x2, gs, gd, wa, wb, wc, bm):
    w_spec = pl.BlockSpec((HID, HID), lambda i: (0, 0))
    m_spec = pl.BlockSpec((bm, HID), lambda i: (i, 0))
    return pl.pallas_call(
        _st4_body,
        grid=(N // bm,),
        in_specs=[pl.BlockSpec((2, bm, 128), lambda i: (0, i, 0)),
                  m_spec, m_spec, w_spec, w_spec, w_spec],
        out_specs=(m_spec, m_spec, m_spec),
        out_shape=(jax.ShapeDtypeStruct((N, HID), jnp.float32),) * 3,
    )(x2, gs, gd, wa, wb, wc)


def _ea_body(ea_ref, w_ref, bias_ref, t_ref, eac_ref):
    acc = (jnp.dot(ea_ref[...], w_ref[...], preferred_element_type=jnp.float32)
           + bias_ref[...])
    t_ref[...] = acc[:, :128]
    eac_ref[...] = acc[:, 128:]


def _ea_mm(ea, w, bias, bm):
    return pl.pallas_call(
        _ea_body,
        grid=(E // bm,),
        in_specs=[pl.BlockSpec((bm, DE), lambda i: (i, 0)),
                  pl.BlockSpec((DE, 384), lambda i: (0, 0)),
                  pl.BlockSpec((1, 384), lambda i: (0, 0))],
        out_specs=(pl.BlockSpec((bm, 128), lambda i: (i, 0)),
                   pl.BlockSpec((bm, HID), lambda i: (i, 0))),
        out_shape=(jax.ShapeDtypeStruct((E, 128), jnp.float32),
                   jax.ShapeDtypeStruct((E, HID), jnp.float32)),
    )(ea, w, bias)


_EC = E // 16
_RCH = 80
_RNCH = _EC // _RCH
_NPOUT = 17408


def _make_gat_sc(has_pq):
    nh = 3584
    nparts = 3
    nhs = nh // 16
    osl = nhs // 2
    scratch = [
        pltpu.VMEM((NP,), jnp.float32),
        pltpu.VMEM((NP,), jnp.float32),
    ]
    if has_pq:
        scratch += [pltpu.VMEM((NP,), jnp.float32),
                    pltpu.VMEM((NP,), jnp.float32)]
    scratch += [
        pltpu.VMEM((_EC,), jnp.int32),
        pltpu.VMEM((_EC,), jnp.int32),
        pltpu.VMEM((_RCH,), jnp.int32),
        pltpu.VMEM((_EC,), jnp.float32),
        pltpu.VMEM((_RCH, 128), jnp.float32),
        pltpu.VMEM((osl, 128), jnp.float32),
        pltpu.VMEM((nhs,), jnp.float32),
        pltpu.VMEM((16, 16), jnp.float32),
        pltpu.VMEM((16,), jnp.float32),
        pltpu.VMEM((128,), jnp.float32),
        pltpu.VMEM_SHARED((nh + 8, 128), jnp.float32),
        pltpu.VMEM_SHARED((nh + 8,), jnp.float32),
        pltpu.VMEM_SHARED((16, 16), jnp.float32),
        pltpu.SemaphoreType.DMA,
    ]

    def body(*args):
        if has_pq:
            (h2_hbm, na_hbm, nd_hbm, ga_hbm, gd_hbm, t_hbm, src_hbm, dst_hbm,
             b2_hbm, out_hbm,
             na_v, nd_v, ga_v, gd_v, src_v, dst_v, idxA, tlw_v,
             rowA, outbuf, sbuf, red_v, misc_v, b_v,
             num_sh, s_sh, red_sh, semA) = args
        else:
            (h2_hbm, na_hbm, nd_hbm, t_hbm, src_hbm, dst_hbm,
             b2_hbm, out_hbm,
             na_v, nd_v, src_v, dst_v, idxA, tlw_v,
             rowA, outbuf, sbuf, red_v, misc_v, b_v,
             num_sh, s_sh, red_sh, semA) = args

        c = lax.axis_index("c")
        sid = lax.axis_index("s")
        ebase = sid * _EC

        pltpu.sync_copy(na_hbm, na_v)
        pltpu.sync_copy(nd_hbm, nd_v)
        if has_pq:
            pltpu.sync_copy(ga_hbm, ga_v)
            pltpu.sync_copy(gd_hbm, gd_v)
        pltpu.sync_copy(src_hbm.at[pl.ds(ebase, _EC)], src_v)
        pltpu.sync_copy(dst_hbm.at[pl.ds(ebase, _EC)], dst_v)
        pltpu.sync_copy(t_hbm.at[pl.ds(ebase, _EC)], tlw_v)
        pltpu.sync_copy(b2_hbm.at[c], b_v)

        def p_logit(i, acc):
            sl = pl.ds(i * 16, 16)
            s16 = src_v[sl]
            d16 = dst_v[sl]
            he = tlw_v[sl]
            if has_pq:
                he = (he + plsc.load_gather(ga_v, [s16])
                      + plsc.load_gather(gd_v, [d16]))
            lo = plsc.load_gather(na_v, [s16]) + plsc.load_gather(nd_v, [d16]) + he
            tlw_v[sl] = _leaky(lo)
            return acc + he

        acc = lax.fori_loop(0, _EC // 16, p_logit,
                            jnp.zeros((16,), jnp.float32))
        misc_v[...] = acc
        pltpu.sync_copy(misc_v, red_sh.at[sid])
        plsc.subcore_barrier()
        pltpu.sync_copy(red_sh, red_v)
        tot = jnp.zeros((16,), jnp.float32)
        for j in range(16):
            tot = tot + red_v[j]
        he_loop = jnp.sum(tot * (1.0 / E))

        def p_w(i, _):
            sl = pl.ds(i * 16, 16)
            d16 = dst_v[sl]
            llv = _leaky(plsc.load_gather(na_v, [d16])
                         + plsc.load_gather(nd_v, [d16]) + he_loop)
            tlw_v[sl] = jnp.exp(tlw_v[sl] - llv)
            return 0

        lax.fori_loop(0, _EC // 16, p_w, 0)

        def mk_chunk(lo, idxbuf, rowbuf):
            def go(cb, cp):
                def mkidx(g, _):
                    d16 = dst_v[pl.ds(cb + g * 16, 16)]
                    inh = (d16 >= lo) & (d16 < lo + nh)
                    idxbuf[pl.ds(g * 16, 16)] = jnp.where(inh, d16 - lo, nh)
                    return 0

                lax.fori_loop(0, _RCH // 16, mkidx, 0)
                pltpu.sync_copy(tlw_v.at[pl.ds(cb, _RCH)],
                                s_sh.at[idxbuf], add=True)
                cp.wait()

                def rowscale(g, _):
                    w16 = tlw_v[pl.ds(cb + g * 16, 16)]
                    for j in range(16):
                        e = g * 16 + j
                        wsc = w16[j]
                        for v in range(8):
                            sl2 = pl.ds(v * 16, 16)
                            rowbuf[e, sl2] = rowbuf[e, sl2] * wsc
                    return 0

                lax.fori_loop(0, _RCH // 16, rowscale, 0)
                pltpu.sync_copy(rowbuf, num_sh.at[idxbuf], add=True)
            return go

        def gather(cb, rowbuf, sem):
            return pltpu.async_copy(
                h2_hbm.at[c].at[src_v.at[pl.ds(cb, _RCH)]], rowbuf, sem)

        for part in range(nparts):
            lo = part * nh
            nbase = sid * nhs

            pltpu.sync_copy(h2_hbm.at[c, pl.ds(lo + nbase, nhs)],
                            num_sh.at[pl.ds(nbase, nhs)])

            def p_one(i, _):
                sbuf[pl.ds(i * 16, 16)] = jnp.full((16,), 1.0, jnp.float32)
                return 0

            lax.fori_loop(0, nhs // 16, p_one, 0)
            pltpu.sync_copy(sbuf, s_sh.at[pl.ds(nbase, nhs)])
            plsc.subcore_barrier()

            goA = mk_chunk(lo, idxA, rowA)

            def p_rows(k, _):
                cb = k * _RCH
                goA(cb, gather(cb, rowA, semA))
                return 0

            lax.fori_loop(0, _RNCH, p_rows, 0)
            plsc.subcore_barrier()

            pltpu.sync_copy(s_sh.at[pl.ds(nbase, nhs)], sbuf)

            def p_inv(i, _):
                sl = pl.ds(i * 16, 16)
                sbuf[sl] = 1.0 / (sbuf[sl] + 1e-16)
                return 0

            lax.fori_loop(0, nhs // 16, p_inv, 0)

            def p_out(jj, _):
                nb2 = jj * osl
                pltpu.sync_copy(num_sh.at[pl.ds(nbase + nb2, osl)], outbuf)

                def orow(g, _):
                    inv16 = sbuf[pl.ds(nb2 + g * 16, 16)]
                    for j in range(16):
                        i = g * 16 + j
                        inv = inv16[j]
                        for v in range(8):
                            sl2 = pl.ds(v * 16, 16)
                            outbuf[i, sl2] = outbuf[i, sl2] * inv + b_v[sl2]
                    return 0

                lax.fori_loop(0, osl // 16, orow, 0)
                pltpu.sync_copy(outbuf,
                                out_hbm.at[c, pl.ds(lo + nbase + nb2, osl)])
                return 0

            lax.fori_loop(0, nhs // osl, p_out, 0)
            plsc.subcore_barrier()

    return functools.partial(
        pl.kernel,
        out_type=jax.ShapeDtypeStruct((2, _NPOUT, 128), jnp.float32),
        mesh=_MESH,
        scratch_types=scratch,
        compiler_params=pltpu.CompilerParams(needs_layout_passes=False),
    )(body)


_gat_sc0 = _make_gat_sc(False)
_gat_sc1 = _make_gat_sc(True)


_EW = E // NW
_ECH = 200
_ENCH = _EW // _ECH


@functools.partial(
    pl.kernel,
    out_type=jax.ShapeDtypeStruct((E, HID), jnp.float32),
    mesh=_MESH,
    scratch_types=[
        pltpu.VMEM((_EW,), jnp.int32),
        pltpu.VMEM((_EW,), jnp.int32),
        pltpu.VMEM((_ECH, HID), jnp.float32),
        pltpu.VMEM((_ECH, HID), jnp.float32),
        pltpu.SemaphoreType.DMA,
        pltpu.SemaphoreType.DMA,
    ],
)
def _e2_assemble(a_hbm, b_hbm, eac_hbm, src_hbm, dst_hbm, out_hbm,
                 src_v, dst_v, bufa, bufb, sema, semb):
    w = lax.axis_index("s") * 2 + lax.axis_index("c")
    ebase = w * _EW
    pltpu.sync_copy(src_hbm.at[pl.ds(ebase, _EW)], src_v)
    pltpu.sync_copy(dst_hbm.at[pl.ds(ebase, _EW)], dst_v)

    def add_into_a(e, _):
        for v in range(HID // 16):
            sl = pl.ds(v * 16, 16)
            bufa[e, sl] = bufa[e, sl] + bufb[e, sl]
        return 0

    def chunk(k, _):
        cb = k * _ECH
        cpa = pltpu.async_copy(a_hbm.at[src_v.at[pl.ds(cb, _ECH)]], bufa, sema)
        cpb = pltpu.async_copy(b_hbm.at[dst_v.at[pl.ds(cb, _ECH)]], bufb, semb)
        cpa.wait()
        cpb.wait()
        lax.fori_loop(0, _ECH, add_into_a, 0)
        pltpu.sync_copy(eac_hbm.at[pl.ds(ebase + cb, _ECH)], bufb)
        lax.fori_loop(0, _ECH, add_into_a, 0)
        pltpu.sync_copy(bufa, out_hbm.at[pl.ds(ebase + cb, _ECH)])
        return 0

    lax.fori_loop(0, _ENCH, chunk, 0)


def _pad_n(v):
    return jnp.pad(v, (0, NP - N))


def kernel(x, edge_index, edge_attr, W0, as0, ad0, We0, ae0, b0,
           W1, as1, ad1, We1, ae1, b1, U0, ub0, U1, ub1):
    src, dst = edge_index[0], edge_index[1]

    we0 = We0 @ ae0
    we1 = We1 @ ae1
    U0a, U0b, U0c = U0[:HID], U0[HID:2 * HID], U0[2 * HID:]
    U1a, U1b, U1c = U1[:HID], U1[HID:2 * HID], U1[2 * HID:]
    C = U0c @ U1c
    dvec = ub0 @ U1c + ub1
    c1 = ub0 @ we1
    v1 = U0c @ we1
    S0 = jnp.zeros((HID, 128), jnp.float32).at[:, 0].set(W0 @ as0).at[:, 1].set(W0 @ ad0)
    Wbig0 = jnp.concatenate([W0, S0], axis=1)
    Svec1 = (jnp.zeros((HID, 128), jnp.float32)
             .at[:, 0].set(W1 @ as1).at[:, 1].set(W1 @ ad1)
             .at[:, 2].set(U0a @ we1).at[:, 3].set(U0b @ we1))
    Wbig1 = jnp.concatenate([W1, U0a, U0b, Svec1], axis=1)
    Wea = (jnp.zeros((DE, 384), jnp.float32)
           .at[:, 0].set(we0).at[:, 1].set(v1).at[:, 128:].set(C))
    bea = jnp.zeros((384,), jnp.float32).at[1].set(c1).at[128:].set(dvec)

    h0s, scal0 = _stage1(x, Wbig0, 1000)
    na0, nd0 = _pad_n(scal0[:, 0]), _pad_n(scal0[:, 1])

    tcols, eac = _ea_mm(edge_attr, Wea, bea[None, :], 4000)
    t0_e, t1_e = tcols[:, 0], tcols[:, 1]

    x1s = _gat_sc0(h0s, na0, nd0, t0_e, src, dst,
                   b0.reshape(2, 128))

    h1s, Gs, Gd, scal1 = _stage3(x1s, Wbig1, 1000)
    na1, nd1 = _pad_n(scal1[:, 0]), _pad_n(scal1[:, 1])
    ga1, gd1 = _pad_n(scal1[:, 2]), _pad_n(scal1[:, 3])

    x2s = _gat_sc1(h1s, na1, nd1, ga1, gd1, t1_e, src, dst,
                   b1.reshape(2, 128))

    A, B, x2 = _stage4(x2s, Gs, Gd, U1a, U1b, U1c, 1000)

    e2 = _e2_assemble(A, B, eac, src, dst)
    return x2, e2
